# baseline (speedup 1.0000x reference)
import functools

import jax
import jax.numpy as jnp
from jax import lax
from jax.experimental import pallas as pl
from jax.experimental.pallas import tpu as pltpu
from jax.experimental.pallas import tpu_sc as plsc

HIDDEN = 128
PACKED = HIDDEN // 2
_INFO = plsc.get_sparse_core_info()
NC = _INFO.num_cores
NS = _INFO.num_subcores
NW = NC * NS

C = 128
NCHUNK = 200
NPAIR = NCHUNK // 2
STAGE = 8
N_TOTAL = NW * NCHUNK * C
HI_MASK = -65536

N_ORG = 100000
N_SMALL = 1000
PBLK = 200
ORG_BLOCKS = N_ORG // PBLK
SMALL_BLOCKS = N_SMALL // PBLK

_SC_PARAMS = pltpu.CompilerParams(
    needs_layout_passes=False, use_tc_tiling_on_sc=False)
_MESH = plsc.VectorSubcoreMesh(core_axis_name="c", subcore_axis_name="s")


def _pack_block(src, dst, b, in_buf, out_buf):
    r0 = b * PBLK
    pltpu.sync_copy(src.at[pl.ds(r0, PBLK)], in_buf)

    def row_body(r, _):
        for jj in range(PACKED // 16):
            a = in_buf[r, pl.ds(jj * 16, 16)]
            bb = in_buf[r, pl.ds(PACKED + jj * 16, 16)]
            w = plsc.bitcast(
                plsc.pack(a, bb, format=plsc.PackFormat.INTERLEAVED),
                jnp.int32)
            out_buf[r, pl.ds(jj * 16, 16)] = w
        return 0

    lax.fori_loop(0, PBLK, row_body, 0)
    pltpu.sync_copy(out_buf, dst.at[pl.ds(r0, PBLK)])


@functools.partial(
    pl.kernel,
    mesh=_MESH,
    out_type=(
        jax.ShapeDtypeStruct((N_SMALL, PACKED), jnp.int32),
        jax.ShapeDtypeStruct((N_ORG, PACKED), jnp.int32),
        jax.ShapeDtypeStruct((N_SMALL, PACKED), jnp.int32),
        jax.ShapeDtypeStruct((N_SMALL, PACKED), jnp.int32),
    ),
    compiler_params=_SC_PARAMS,
    scratch_types=[
        pltpu.VMEM((PBLK, HIDDEN), jnp.float32),
        pltpu.VMEM((PBLK, PACKED), jnp.int32),
    ],
)
def _pack_tables(spec_t, org_t, abx_t, intp_t,
                 spec_p, org_p, abx_p, intp_p, in_buf, out_buf):
    wid = lax.axis_index("s") * NC + lax.axis_index("c")

    @pl.when(wid < SMALL_BLOCKS)
    def _():
        _pack_block(spec_t, spec_p, wid, in_buf, out_buf)

    @pl.when(jnp.logical_and(wid >= 8, wid < 8 + SMALL_BLOCKS))
    def _():
        _pack_block(abx_t, abx_p, wid - 8, in_buf, out_buf)

    @pl.when(jnp.logical_and(wid >= 16, wid < 16 + SMALL_BLOCKS))
    def _():
        _pack_block(intp_t, intp_p, wid - 16, in_buf, out_buf)

    def org_body(k, _):
        b = wid + NW * k

        @pl.when(b < ORG_BLOCKS)
        def _():
            _pack_block(org_t, org_p, b, in_buf, out_buf)

        return 0

    lax.fori_loop(0, (ORG_BLOCKS + NW - 1) // NW, org_body, 0)


def _unpack_sum_row(bufs, out_v, r):
    for jj in range(PACKED // 16):
        sl = pl.ds(jj * 16, 16)
        w0 = bufs[0][r, sl]
        w1 = bufs[1][r, sl]
        w2 = bufs[2][r, sl]
        w3 = bufs[3][r, sl]
        lo = (plsc.bitcast(lax.shift_left(w0, 16), jnp.float32)
              + plsc.bitcast(lax.shift_left(w1, 16), jnp.float32)
              + plsc.bitcast(lax.shift_left(w2, 16), jnp.float32)
              + plsc.bitcast(lax.shift_left(w3, 16), jnp.float32))
        hi = (plsc.bitcast(w0 & HI_MASK, jnp.float32)
              + plsc.bitcast(w1 & HI_MASK, jnp.float32)
              + plsc.bitcast(w2 & HI_MASK, jnp.float32)
              + plsc.bitcast(w3 & HI_MASK, jnp.float32))
        out_v[r, sl] = lo
        out_v[r, pl.ds(PACKED + jj * 16, 16)] = hi


def _sc_body(idx_hbm, tables, out_h, idx_v, bufs_a, bufs_b, out_a, out_b,
             gsem_a, gsem_b, isem, ssem_a, ssem_b):
    wid = lax.axis_index("s") * NC + lax.axis_index("c")
    comb_v, org_v, abx_v, intp_v, spec_v = idx_v

    def fire_idx_stage(st, slot8):
        for t in range(2):
            pltpu.async_copy(
                idx_hbm[t].at[wid, pl.ds(st * STAGE, STAGE)],
                idx_v[t].at[pl.ds(slot8 * STAGE, STAGE)], isem)

    def wait_idx_stage():
        for t in range(2):
            pltpu.make_async_copy(
                idx_hbm[t].at[wid, pl.ds(0, STAGE)],
                idx_v[t].at[pl.ds(0, STAGE)], isem).wait()

    def derive_idx_stage(slot8):
        base = slot8 * STAGE

        def row_body(r, _):
            row = base + r
            for v in range(C // 16):
                sl = pl.ds(v * 16, 16)
                cm = comb_v[row, sl]
                spec_v[row, sl] = cm & 1023
                abx_v[row, sl] = lax.shift_right_logical(cm, 10) & 1023
                intp_v[row, sl] = lax.shift_right_logical(cm, 20) & 1023
                org_v[row, sl] = org_v[row, sl] - N_SMALL
            return 0

        lax.fori_loop(0, STAGE, row_body, 0)

    lists = (spec_v, org_v, abx_v, intp_v)

    def fire_gathers(j, bufs, gsem):
        slot = lax.rem(j, 2 * STAGE)
        for t in range(4):
            pltpu.async_copy(tables[t].at[lists[t].at[slot]], bufs[t], gsem)

    def wait_gathers(bufs, gsem):
        for t in range(4):
            pltpu.make_async_copy(
                tables[t].at[lists[t].at[0]], bufs[t], gsem).wait()

    def wait_store(out_v, ssem):
        pltpu.make_async_copy(out_v, out_h.at[pl.ds(0, C)], ssem).wait()

    fire_idx_stage(0, 0)
    wait_idx_stage()
    derive_idx_stage(0)
    fire_gathers(0, bufs_a, gsem_a)
    fire_gathers(1, bufs_b, gsem_b)

    def pair_body(pj, _):
        j0 = 2 * pj
        st = pj // 4

        @pl.when(jnp.logical_and(lax.rem(pj, 4) == 0, pj < NPAIR - 4))
        def _():
            fire_idx_stage(st + 1, lax.rem(st + 1, 2))

        def half(j, bufs, gsem, out_v, ssem, other_first):
            wait_gathers(bufs, gsem)

            @pl.when(pj > 0)
            def _():
                wait_store(out_v, ssem)

            def row_body(r, _):
                _unpack_sum_row(bufs, out_v, r)
                return 0

            lax.fori_loop(0, C, row_body, 0)
            row0 = (wid * NCHUNK + j) * C
            pltpu.async_copy(out_v, out_h.at[pl.ds(row0, C)], ssem)

            @pl.when(pj < NPAIR - 1)
            def _():
                if other_first:
                    @pl.when(lax.rem(pj, 4) == 3)
                    def _():
                        wait_idx_stage()
                        derive_idx_stage(lax.rem(st + 1, 2))
                fire_gathers(j + 2, bufs, gsem)

        half(j0, bufs_a, gsem_a, out_a, ssem_a, True)
        half(j0 + 1, bufs_b, gsem_b, out_b, ssem_b, False)
        return 0

    lax.fori_loop(0, NPAIR, pair_body, 0)
    wait_store(out_a, ssem_a)
    wait_store(out_b, ssem_b)


@functools.partial(
    pl.kernel,
    mesh=_MESH,
    out_type=jax.ShapeDtypeStruct((N_TOTAL, HIDDEN), jnp.float32),
    compiler_params=_SC_PARAMS,
    scratch_types=[
        [pltpu.VMEM((2 * STAGE, C), jnp.int32)] * 5,
        [pltpu.VMEM((C, PACKED), jnp.int32)] * 4,
        [pltpu.VMEM((C, PACKED), jnp.int32)] * 4,
        pltpu.VMEM((C, HIDDEN), jnp.float32),
        pltpu.VMEM((C, HIDDEN), jnp.float32),
        pltpu.SemaphoreType.DMA,
        pltpu.SemaphoreType.DMA,
        pltpu.SemaphoreType.DMA,
        pltpu.SemaphoreType.DMA,
        pltpu.SemaphoreType.DMA,
    ],
)
def _embed_sum(comb_idx, org_idx,
               spec_t, org_t, abx_t, intp_t,
               out_h, idx_v, bufs_a, bufs_b, out_a, out_b,
               gsem_a, gsem_b, isem, ssem_a, ssem_b):
    _sc_body((comb_idx, org_idx),
             (spec_t, org_t, abx_t, intp_t),
             out_h, idx_v, bufs_a, bufs_b, out_a, out_b,
             gsem_a, gsem_b, isem, ssem_a, ssem_b)


def kernel(specimen_ids, organism_ids, antibiotic_ids, interpretation_ids,
           specimen_table, organism_table, antibiotic_table, interpretation_table):
    batch, hist = specimen_ids.shape
    spec_p, org_p, abx_p, intp_p = _pack_tables(
        specimen_table, organism_table, antibiotic_table, interpretation_table)
    shp = (NW, NCHUNK, C)
    comb = (specimen_ids | (antibiotic_ids << 10)
            | (interpretation_ids << 20)).reshape(shp)
    orga = (organism_ids + N_SMALL).reshape(shp)
    out = _embed_sum(comb, orga, spec_p, org_p, abx_p, intp_p)
    return out.reshape(batch, hist, HIDDEN)


# speedup vs baseline: 1.7460x; 1.7460x over previous
import functools

import jax
import jax.numpy as jnp
from jax import lax
from jax.experimental import pallas as pl
from jax.experimental.pallas import tpu as pltpu
from jax.experimental.pallas import tpu_sc as plsc

HIDDEN = 128
PACKED = HIDDEN // 2
_INFO = plsc.get_sparse_core_info()
NC = _INFO.num_cores
NS = _INFO.num_subcores
NW = NC * NS

C = 128
NCHUNK = 200
NPAIR = NCHUNK // 2
STAGE = 8
N_TOTAL = NW * NCHUNK * C
HI_MASK = -65536

N_ORG = 100000
N_SMALL = 1000
PBLK = 200
ORG_BLOCKS = N_ORG // PBLK
SMALL_BLOCKS = N_SMALL // PBLK

_SC_PARAMS = pltpu.CompilerParams(
    needs_layout_passes=False, use_tc_tiling_on_sc=False)
_MESH = plsc.VectorSubcoreMesh(core_axis_name="c", subcore_axis_name="s")


def _pack_block(src, dst, b, in_buf, out_buf):
    r0 = b * PBLK
    pltpu.sync_copy(src.at[pl.ds(r0, PBLK)], in_buf)

    def row_body(r, _):
        for jj in range(PACKED // 16):
            a = in_buf[r, pl.ds(jj * 16, 16)]
            bb = in_buf[r, pl.ds(PACKED + jj * 16, 16)]
            w = plsc.bitcast(
                plsc.pack(a, bb, format=plsc.PackFormat.INTERLEAVED),
                jnp.int32)
            out_buf[r, pl.ds(jj * 16, 16)] = w
        return 0

    lax.fori_loop(0, PBLK, row_body, 0)
    pltpu.sync_copy(out_buf, dst.at[pl.ds(r0, PBLK)])


@functools.partial(
    pl.kernel,
    mesh=_MESH,
    out_type=(
        jax.ShapeDtypeStruct((N_SMALL, PACKED), jnp.int32),
        jax.ShapeDtypeStruct((N_ORG, PACKED), jnp.int32),
        jax.ShapeDtypeStruct((N_SMALL, PACKED), jnp.int32),
        jax.ShapeDtypeStruct((N_SMALL, PACKED), jnp.int32),
    ),
    compiler_params=_SC_PARAMS,
    scratch_types=[
        pltpu.VMEM((PBLK, HIDDEN), jnp.float32),
        pltpu.VMEM((PBLK, PACKED), jnp.int32),
    ],
)
def _pack_tables(spec_t, org_t, abx_t, intp_t,
                 spec_p, org_p, abx_p, intp_p, in_buf, out_buf):
    wid = lax.axis_index("s") * NC + lax.axis_index("c")

    @pl.when(wid < SMALL_BLOCKS)
    def _():
        _pack_block(spec_t, spec_p, wid, in_buf, out_buf)

    @pl.when(jnp.logical_and(wid >= 8, wid < 8 + SMALL_BLOCKS))
    def _():
        _pack_block(abx_t, abx_p, wid - 8, in_buf, out_buf)

    @pl.when(jnp.logical_and(wid >= 16, wid < 16 + SMALL_BLOCKS))
    def _():
        _pack_block(intp_t, intp_p, wid - 16, in_buf, out_buf)

    def org_body(k, _):
        b = wid + NW * k

        @pl.when(b < ORG_BLOCKS)
        def _():
            _pack_block(org_t, org_p, b, in_buf, out_buf)

        return 0

    lax.fori_loop(0, (ORG_BLOCKS + NW - 1) // NW, org_body, 0)


def _unpack_sum_row(bufs, out_v, r):
    for jj in range(PACKED // 16):
        sl = pl.ds(jj * 16, 16)
        w0 = bufs[0][r, sl]
        w1 = bufs[1][r, sl]
        w2 = bufs[2][r, sl]
        w3 = bufs[3][r, sl]
        lo = (plsc.bitcast(lax.shift_left(w0, 16), jnp.float32)
              + plsc.bitcast(lax.shift_left(w1, 16), jnp.float32)
              + plsc.bitcast(lax.shift_left(w2, 16), jnp.float32)
              + plsc.bitcast(lax.shift_left(w3, 16), jnp.float32))
        hi = (plsc.bitcast(w0 & HI_MASK, jnp.float32)
              + plsc.bitcast(w1 & HI_MASK, jnp.float32)
              + plsc.bitcast(w2 & HI_MASK, jnp.float32)
              + plsc.bitcast(w3 & HI_MASK, jnp.float32))
        out_v[r, sl] = lo
        out_v[r, pl.ds(PACKED + jj * 16, 16)] = hi


def _sc_body(idx_hbm, tables, out_h, idx_v, bufs_a, bufs_b, out_a, out_b,
             gsem_a, gsem_b, isem, ssem_a, ssem_b):
    wid = lax.axis_index("s") * NC + lax.axis_index("c")
    comb_v, org_v, abx_v, intp_v, spec_v = idx_v

    def fire_idx_stage(st, slot8):
        for t in range(2):
            pltpu.async_copy(
                idx_hbm[t].at[wid, pl.ds(st * STAGE, STAGE)],
                idx_v[t].at[pl.ds(slot8 * STAGE, STAGE)], isem)

    def wait_idx_stage():
        for t in range(2):
            pltpu.make_async_copy(
                idx_hbm[t].at[wid, pl.ds(0, STAGE)],
                idx_v[t].at[pl.ds(0, STAGE)], isem).wait()

    def derive_idx_stage(slot8):
        base = slot8 * STAGE

        def row_body(r, _):
            row = base + r
            for v in range(C // 16):
                sl = pl.ds(v * 16, 16)
                cm = comb_v[row, sl]
                spec_v[row, sl] = cm & 1023
                abx_v[row, sl] = lax.shift_right_logical(cm, 10) & 1023
                intp_v[row, sl] = lax.shift_right_logical(cm, 20) & 1023
                org_v[row, sl] = org_v[row, sl] - N_SMALL
            return 0

        lax.fori_loop(0, STAGE, row_body, 0)

    lists = (spec_v, org_v, abx_v, intp_v)

    def fire_gathers(j, bufs, gsem):
        slot = lax.rem(j, 2 * STAGE)
        for t in range(4):
            pltpu.async_copy(tables[t].at[lists[t].at[slot]], bufs[t], gsem)

    def wait_gathers(bufs, gsem):
        for t in range(4):
            pltpu.make_async_copy(
                tables[t].at[lists[t].at[0]], bufs[t], gsem).wait()

    def wait_store(out_v, ssem):
        pltpu.make_async_copy(out_v, out_h.at[pl.ds(0, C)], ssem).wait()

    fire_idx_stage(0, 0)
    wait_idx_stage()
    derive_idx_stage(0)
    fire_gathers(0, bufs_a, gsem_a)
    fire_gathers(1, bufs_b, gsem_b)

    def pair_body(pj, _):
        j0 = 2 * pj
        st = pj // 4

        @pl.when(jnp.logical_and(lax.rem(pj, 4) == 0, pj < NPAIR - 4))
        def _():
            fire_idx_stage(st + 1, lax.rem(st + 1, 2))

        def half(j, bufs, gsem, out_v, ssem, other_first):
            wait_gathers(bufs, gsem)

            @pl.when(pj > 0)
            def _():
                wait_store(out_v, ssem)

            def row_body(r, _):
                _unpack_sum_row(bufs, out_v, r)
                return 0

            lax.fori_loop(0, C, row_body, 0)
            row0 = (wid * NCHUNK + j) * C
            pltpu.async_copy(out_v, out_h.at[pl.ds(row0, C)], ssem)

            @pl.when(pj < NPAIR - 1)
            def _():
                if other_first:
                    @pl.when(lax.rem(pj, 4) == 3)
                    def _():
                        wait_idx_stage()
                        derive_idx_stage(lax.rem(st + 1, 2))
                fire_gathers(j + 2, bufs, gsem)

        half(j0, bufs_a, gsem_a, out_a, ssem_a, True)
        half(j0 + 1, bufs_b, gsem_b, out_b, ssem_b, False)
        return 0

    lax.fori_loop(0, NPAIR, pair_body, 0)
    wait_store(out_a, ssem_a)
    wait_store(out_b, ssem_b)


@functools.partial(
    pl.kernel,
    mesh=_MESH,
    out_type=jax.ShapeDtypeStruct((N_TOTAL, HIDDEN), jnp.float32),
    compiler_params=_SC_PARAMS,
    scratch_types=[
        [pltpu.VMEM((2 * STAGE, C), jnp.int32)] * 5,
        [pltpu.VMEM((C, PACKED), jnp.int32)] * 4,
        [pltpu.VMEM((C, PACKED), jnp.int32)] * 4,
        pltpu.VMEM((C, HIDDEN), jnp.float32),
        pltpu.VMEM((C, HIDDEN), jnp.float32),
        pltpu.SemaphoreType.DMA,
        pltpu.SemaphoreType.DMA,
        pltpu.SemaphoreType.DMA,
        pltpu.SemaphoreType.DMA,
        pltpu.SemaphoreType.DMA,
    ],
)
def _embed_sum(comb_idx, org_idx,
               spec_t, org_t, abx_t, intp_t,
               out_h, idx_v, bufs_a, bufs_b, out_a, out_b,
               gsem_a, gsem_b, isem, ssem_a, ssem_b):
    _sc_body((comb_idx, org_idx),
             (spec_t, org_t, abx_t, intp_t),
             out_h, idx_v, bufs_a, bufs_b, out_a, out_b,
             gsem_a, gsem_b, isem, ssem_a, ssem_b)


def kernel(specimen_ids, organism_ids, antibiotic_ids, interpretation_ids,
           specimen_table, organism_table, antibiotic_table, interpretation_table):
    batch, hist = specimen_ids.shape
    spec_p, org_p, abx_p, intp_p = _pack_tables(
        specimen_table, organism_table, antibiotic_table, interpretation_table)
    shp = (NW, NCHUNK, C)
    comb = (specimen_ids | (antibiotic_ids << 10)
            | (interpretation_ids << 20)).T.reshape(shp)
    orga = (organism_ids + N_SMALL).T.reshape(shp)
    out = _embed_sum(comb, orga, spec_p, org_p, abx_p, intp_p)
    return out.reshape(hist, batch, HIDDEN).transpose(1, 0, 2)


# speedup vs baseline: 1.7651x; 1.0109x over previous
import functools

import jax
import jax.numpy as jnp
from jax import lax
from jax.experimental import pallas as pl
from jax.experimental.pallas import tpu as pltpu
from jax.experimental.pallas import tpu_sc as plsc

HIDDEN = 128
PACKED = HIDDEN // 2
_INFO = plsc.get_sparse_core_info()
NC = _INFO.num_cores
NS = _INFO.num_subcores
NW = NC * NS

C = 128
NCHUNK = 200
NPAIR = NCHUNK // 2
STAGE = 8
N_TOTAL = NW * NCHUNK * C
HI_MASK = -65536

N_ORG = 100000
N_SMALL = 1000
PBLK = 400
PBLK_S = 200
ORG_BLOCKS = N_ORG // PBLK
SMALL_BLOCKS = N_SMALL // PBLK_S

_SC_PARAMS = pltpu.CompilerParams(
    needs_layout_passes=False, use_tc_tiling_on_sc=False)
_MESH = plsc.VectorSubcoreMesh(core_axis_name="c", subcore_axis_name="s")


def _pack_block(src, dst, b, in_buf, out_buf, nrows):
    r0 = b * nrows
    pltpu.sync_copy(src.at[pl.ds(r0, nrows)], in_buf.at[pl.ds(0, nrows)])

    def row_body(r, _):
        for jj in range(PACKED // 16):
            a = in_buf[r, pl.ds(jj * 16, 16)]
            bb = in_buf[r, pl.ds(PACKED + jj * 16, 16)]
            w = plsc.bitcast(
                plsc.pack(a, bb, format=plsc.PackFormat.INTERLEAVED),
                jnp.int32)
            out_buf[r, pl.ds(jj * 16, 16)] = w
        return 0

    lax.fori_loop(0, nrows, row_body, 0)
    pltpu.sync_copy(out_buf.at[pl.ds(0, nrows)], dst.at[pl.ds(r0, nrows)])


@functools.partial(
    pl.kernel,
    mesh=_MESH,
    out_type=(
        jax.ShapeDtypeStruct((N_SMALL, PACKED), jnp.int32),
        jax.ShapeDtypeStruct((N_ORG, PACKED), jnp.int32),
        jax.ShapeDtypeStruct((N_SMALL, PACKED), jnp.int32),
        jax.ShapeDtypeStruct((N_SMALL, PACKED), jnp.int32),
    ),
    compiler_params=_SC_PARAMS,
    scratch_types=[
        pltpu.VMEM((PBLK, HIDDEN), jnp.float32),
        pltpu.VMEM((PBLK, PACKED), jnp.int32),
    ],

)
def _pack_tables(spec_t, org_t, abx_t, intp_t,
                 spec_p, org_p, abx_p, intp_p, in_buf, out_buf):
    wid = lax.axis_index("s") * NC + lax.axis_index("c")

    @pl.when(wid < SMALL_BLOCKS)
    def _():
        _pack_block(spec_t, spec_p, wid, in_buf, out_buf, PBLK_S)

    @pl.when(jnp.logical_and(wid >= 8, wid < 8 + SMALL_BLOCKS))
    def _():
        _pack_block(abx_t, abx_p, wid - 8, in_buf, out_buf, PBLK_S)

    @pl.when(jnp.logical_and(wid >= 16, wid < 16 + SMALL_BLOCKS))
    def _():
        _pack_block(intp_t, intp_p, wid - 16, in_buf, out_buf, PBLK_S)

    def org_body(k, _):
        b = wid + NW * k

        @pl.when(b < ORG_BLOCKS)
        def _():
            _pack_block(org_t, org_p, b, in_buf, out_buf, PBLK)

        return 0

    lax.fori_loop(0, (ORG_BLOCKS + NW - 1) // NW, org_body, 0)


def _unpack_sum_row(bufs, out_v, r):
    for jj in range(PACKED // 16):
        sl = pl.ds(jj * 16, 16)
        w0 = bufs[0][r, sl]
        w1 = bufs[1][r, sl]
        w2 = bufs[2][r, sl]
        w3 = bufs[3][r, sl]
        lo = (plsc.bitcast(lax.shift_left(w0, 16), jnp.float32)
              + plsc.bitcast(lax.shift_left(w1, 16), jnp.float32)
              + plsc.bitcast(lax.shift_left(w2, 16), jnp.float32)
              + plsc.bitcast(lax.shift_left(w3, 16), jnp.float32))
        hi = (plsc.bitcast(w0 & HI_MASK, jnp.float32)
              + plsc.bitcast(w1 & HI_MASK, jnp.float32)
              + plsc.bitcast(w2 & HI_MASK, jnp.float32)
              + plsc.bitcast(w3 & HI_MASK, jnp.float32))
        out_v[r, sl] = lo
        out_v[r, pl.ds(PACKED + jj * 16, 16)] = hi


def _sc_body(idx_hbm, tables, out_h, idx_v, bufs_a, bufs_b, out_a, out_b,
             gsem_a, gsem_b, isem, ssem_a, ssem_b):
    wid = lax.axis_index("s") * NC + lax.axis_index("c")
    comb_v, org_v, abx_v, intp_v, spec_v = idx_v

    def fire_idx_stage(st, slot8):
        for t in range(2):
            pltpu.async_copy(
                idx_hbm[t].at[wid, pl.ds(st * STAGE, STAGE)],
                idx_v[t].at[pl.ds(slot8 * STAGE, STAGE)], isem)

    def wait_idx_stage():
        for t in range(2):
            pltpu.make_async_copy(
                idx_hbm[t].at[wid, pl.ds(0, STAGE)],
                idx_v[t].at[pl.ds(0, STAGE)], isem).wait()

    def derive_idx_stage(slot8):
        base = slot8 * STAGE

        def row_body(r, _):
            row = base + r
            for v in range(C // 16):
                sl = pl.ds(v * 16, 16)
                cm = comb_v[row, sl]
                spec_v[row, sl] = cm & 1023
                abx_v[row, sl] = lax.shift_right_logical(cm, 10) & 1023
                intp_v[row, sl] = lax.shift_right_logical(cm, 20) & 1023
                org_v[row, sl] = org_v[row, sl] - N_SMALL
            return 0

        lax.fori_loop(0, STAGE, row_body, 0)

    lists = (spec_v, org_v, abx_v, intp_v)

    def fire_gathers(j, bufs, gsem):
        slot = lax.rem(j, 2 * STAGE)
        for t in range(4):
            pltpu.async_copy(tables[t].at[lists[t].at[slot]], bufs[t], gsem)

    def wait_gathers(bufs, gsem):
        for t in range(4):
            pltpu.make_async_copy(
                tables[t].at[lists[t].at[0]], bufs[t], gsem).wait()

    def wait_store(out_v, ssem):
        pltpu.make_async_copy(out_v, out_h.at[pl.ds(0, C)], ssem).wait()

    fire_idx_stage(0, 0)
    wait_idx_stage()
    derive_idx_stage(0)
    fire_gathers(0, bufs_a, gsem_a)
    fire_gathers(1, bufs_b, gsem_b)

    def pair_body(pj, _):
        j0 = 2 * pj
        st = pj // 4

        @pl.when(jnp.logical_and(lax.rem(pj, 4) == 0, pj < NPAIR - 4))
        def _():
            fire_idx_stage(st + 1, lax.rem(st + 1, 2))

        def half(j, bufs, gsem, out_v, ssem, other_first):
            wait_gathers(bufs, gsem)

            @pl.when(pj > 0)
            def _():
                wait_store(out_v, ssem)

            def row_body(r, _):
                _unpack_sum_row(bufs, out_v, r)
                return 0

            lax.fori_loop(0, C, row_body, 0)
            row0 = (wid * NCHUNK + j) * C
            pltpu.async_copy(out_v, out_h.at[pl.ds(row0, C)], ssem)

            @pl.when(pj < NPAIR - 1)
            def _():
                if other_first:
                    @pl.when(lax.rem(pj, 4) == 3)
                    def _():
                        wait_idx_stage()
                        derive_idx_stage(lax.rem(st + 1, 2))
                fire_gathers(j + 2, bufs, gsem)

        half(j0, bufs_a, gsem_a, out_a, ssem_a, True)
        half(j0 + 1, bufs_b, gsem_b, out_b, ssem_b, False)
        return 0

    lax.fori_loop(0, NPAIR, pair_body, 0)
    wait_store(out_a, ssem_a)
    wait_store(out_b, ssem_b)


@functools.partial(
    pl.kernel,
    mesh=_MESH,
    out_type=jax.ShapeDtypeStruct((N_TOTAL, HIDDEN), jnp.float32),
    compiler_params=_SC_PARAMS,
    scratch_types=[
        [pltpu.VMEM((2 * STAGE, C), jnp.int32)] * 5,
        [pltpu.VMEM((C, PACKED), jnp.int32)] * 4,
        [pltpu.VMEM((C, PACKED), jnp.int32)] * 4,
        pltpu.VMEM((C, HIDDEN), jnp.float32),
        pltpu.VMEM((C, HIDDEN), jnp.float32),
        pltpu.SemaphoreType.DMA,
        pltpu.SemaphoreType.DMA,
        pltpu.SemaphoreType.DMA,
        pltpu.SemaphoreType.DMA,
        pltpu.SemaphoreType.DMA,
    ],
)
def _embed_sum(comb_idx, org_idx,
               spec_t, org_t, abx_t, intp_t,
               out_h, idx_v, bufs_a, bufs_b, out_a, out_b,
               gsem_a, gsem_b, isem, ssem_a, ssem_b):
    _sc_body((comb_idx, org_idx),
             (spec_t, org_t, abx_t, intp_t),
             out_h, idx_v, bufs_a, bufs_b, out_a, out_b,
             gsem_a, gsem_b, isem, ssem_a, ssem_b)


def kernel(specimen_ids, organism_ids, antibiotic_ids, interpretation_ids,
           specimen_table, organism_table, antibiotic_table, interpretation_table):
    batch, hist = specimen_ids.shape
    spec_p, org_p, abx_p, intp_p = _pack_tables(
        specimen_table, organism_table, antibiotic_table, interpretation_table)
    shp = (NW, NCHUNK, C)
    comb = (specimen_ids | (antibiotic_ids << 10)
            | (interpretation_ids << 20)).T.reshape(shp)
    orga = (organism_ids + N_SMALL).T.reshape(shp)
    out = _embed_sum(comb, orga, spec_p, org_p, abx_p, intp_p)
    return out.reshape(hist, batch, HIDDEN).transpose(1, 0, 2)


# speedup vs baseline: 1.8940x; 1.0730x over previous
import functools

import jax
import jax.numpy as jnp
from jax import lax
from jax.experimental import pallas as pl
from jax.experimental.pallas import tpu as pltpu
from jax.experimental.pallas import tpu_sc as plsc

HIDDEN = 128
PACKED = HIDDEN // 2
_INFO = plsc.get_sparse_core_info()
NC = _INFO.num_cores
NS = _INFO.num_subcores
NW = NC * NS

C = 128
NCHUNK = 200
NPAIR = NCHUNK // 2
STAGE = 8
N_TOTAL = NW * NCHUNK * C
HI_MASK = -65536

N_ORG = 100000
N_SMALL = 1000
PBLK = 400
PBLK_S = 200
ORG_BLOCKS = N_ORG // PBLK
SMALL_BLOCKS = N_SMALL // PBLK_S

_SC_PARAMS = pltpu.CompilerParams(
    needs_layout_passes=False, use_tc_tiling_on_sc=False)
_MESH = plsc.VectorSubcoreMesh(core_axis_name="c", subcore_axis_name="s")


def _pack_block(src, dst, b, in_buf, out_buf, nrows):
    r0 = b * nrows
    pltpu.sync_copy(src.at[pl.ds(r0, nrows)], in_buf.at[pl.ds(0, nrows)])

    def row_body(r, _):
        for jj in range(PACKED // 16):
            a = in_buf[r, pl.ds(jj * 16, 16)]
            bb = in_buf[r, pl.ds(PACKED + jj * 16, 16)]
            w = plsc.bitcast(
                plsc.pack(a, bb, format=plsc.PackFormat.INTERLEAVED),
                jnp.int32)
            out_buf[r, pl.ds(jj * 16, 16)] = w
        return 0

    lax.fori_loop(0, nrows, row_body, 0)
    pltpu.sync_copy(out_buf.at[pl.ds(0, nrows)], dst.at[pl.ds(r0, nrows)])


@functools.partial(
    pl.kernel,
    mesh=_MESH,
    out_type=(
        jax.ShapeDtypeStruct((N_SMALL, PACKED), jnp.int32),
        jax.ShapeDtypeStruct((N_SMALL, PACKED), jnp.int32),
        jax.ShapeDtypeStruct((N_SMALL, PACKED), jnp.int32),
    ),
    compiler_params=_SC_PARAMS,
    scratch_types=[
        pltpu.VMEM((PBLK_S, HIDDEN), jnp.float32),
        pltpu.VMEM((PBLK_S, PACKED), jnp.int32),
    ],
)
def _pack_tables(spec_t, abx_t, intp_t,
                 spec_p, abx_p, intp_p, in_buf, out_buf):
    wid = lax.axis_index("s") * NC + lax.axis_index("c")

    @pl.when(wid < SMALL_BLOCKS)
    def _():
        _pack_block(spec_t, spec_p, wid, in_buf, out_buf, PBLK_S)

    @pl.when(jnp.logical_and(wid >= 8, wid < 8 + SMALL_BLOCKS))
    def _():
        _pack_block(abx_t, abx_p, wid - 8, in_buf, out_buf, PBLK_S)

    @pl.when(jnp.logical_and(wid >= 16, wid < 16 + SMALL_BLOCKS))
    def _():
        _pack_block(intp_t, intp_p, wid - 16, in_buf, out_buf, PBLK_S)


def _unpack_sum_row(bufs, out_v, r):
    for jj in range(PACKED // 16):
        sl = pl.ds(jj * 16, 16)
        hsl = pl.ds(PACKED + jj * 16, 16)
        w0 = bufs[0][r, sl]
        w2 = bufs[2][r, sl]
        w3 = bufs[3][r, sl]
        lo = (plsc.bitcast(lax.shift_left(w0, 16), jnp.float32)
              + bufs[1][r, sl]
              + plsc.bitcast(lax.shift_left(w2, 16), jnp.float32)
              + plsc.bitcast(lax.shift_left(w3, 16), jnp.float32))
        hi = (plsc.bitcast(w0 & HI_MASK, jnp.float32)
              + bufs[1][r, hsl]
              + plsc.bitcast(w2 & HI_MASK, jnp.float32)
              + plsc.bitcast(w3 & HI_MASK, jnp.float32))
        out_v[r, sl] = lo
        out_v[r, hsl] = hi


def _sc_body(idx_hbm, tables, out_h, idx_v, bufs_a, bufs_b, out_a, out_b,
             gsem_a, gsem_b, isem, ssem_a, ssem_b):
    wid = lax.axis_index("s") * NC + lax.axis_index("c")
    comb_v, org_v, abx_v, intp_v, spec_v = idx_v

    def fire_idx_stage(st, slot8):
        for t in range(2):
            pltpu.async_copy(
                idx_hbm[t].at[wid, pl.ds(st * STAGE, STAGE)],
                idx_v[t].at[pl.ds(slot8 * STAGE, STAGE)], isem)

    def wait_idx_stage():
        for t in range(2):
            pltpu.make_async_copy(
                idx_hbm[t].at[wid, pl.ds(0, STAGE)],
                idx_v[t].at[pl.ds(0, STAGE)], isem).wait()

    def derive_idx_stage(slot8):
        base = slot8 * STAGE

        def row_body(r, _):
            row = base + r
            for v in range(C // 16):
                sl = pl.ds(v * 16, 16)
                cm = comb_v[row, sl]
                spec_v[row, sl] = cm & 1023
                abx_v[row, sl] = lax.shift_right_logical(cm, 10) & 1023
                intp_v[row, sl] = lax.shift_right_logical(cm, 20) & 1023
                org_v[row, sl] = org_v[row, sl] - N_SMALL
            return 0

        lax.fori_loop(0, STAGE, row_body, 0)

    lists = (spec_v, org_v, abx_v, intp_v)

    def fire_gathers(j, bufs, gsem):
        slot = lax.rem(j, 2 * STAGE)
        for t in range(4):
            pltpu.async_copy(tables[t].at[lists[t].at[slot]], bufs[t], gsem)

    def wait_gathers(bufs, gsem):
        for t in range(4):
            pltpu.make_async_copy(
                tables[t].at[lists[t].at[0]], bufs[t], gsem).wait()

    def wait_store(out_v, ssem):
        pltpu.make_async_copy(out_v, out_h.at[pl.ds(0, C)], ssem).wait()

    fire_idx_stage(0, 0)
    wait_idx_stage()
    derive_idx_stage(0)
    fire_gathers(0, bufs_a, gsem_a)
    fire_gathers(1, bufs_b, gsem_b)

    def pair_body(pj, _):
        j0 = 2 * pj
        st = pj // 4

        @pl.when(jnp.logical_and(lax.rem(pj, 4) == 0, pj < NPAIR - 4))
        def _():
            fire_idx_stage(st + 1, lax.rem(st + 1, 2))

        def half(j, bufs, gsem, out_v, ssem, other_first):
            wait_gathers(bufs, gsem)

            @pl.when(pj > 0)
            def _():
                wait_store(out_v, ssem)

            def row_body(r, _):
                _unpack_sum_row(bufs, out_v, r)
                return 0

            lax.fori_loop(0, C, row_body, 0)
            row0 = (wid * NCHUNK + j) * C
            pltpu.async_copy(out_v, out_h.at[pl.ds(row0, C)], ssem)

            @pl.when(pj < NPAIR - 1)
            def _():
                if other_first:
                    @pl.when(lax.rem(pj, 4) == 3)
                    def _():
                        wait_idx_stage()
                        derive_idx_stage(lax.rem(st + 1, 2))
                fire_gathers(j + 2, bufs, gsem)

        half(j0, bufs_a, gsem_a, out_a, ssem_a, True)
        half(j0 + 1, bufs_b, gsem_b, out_b, ssem_b, False)
        return 0

    lax.fori_loop(0, NPAIR, pair_body, 0)
    wait_store(out_a, ssem_a)
    wait_store(out_b, ssem_b)


@functools.partial(
    pl.kernel,
    mesh=_MESH,
    out_type=jax.ShapeDtypeStruct((N_TOTAL, HIDDEN), jnp.float32),
    compiler_params=_SC_PARAMS,
    scratch_types=[
        [pltpu.VMEM((2 * STAGE, C), jnp.int32)] * 5,
        [pltpu.VMEM((C, PACKED), jnp.int32),
         pltpu.VMEM((C, HIDDEN), jnp.float32),
         pltpu.VMEM((C, PACKED), jnp.int32),
         pltpu.VMEM((C, PACKED), jnp.int32)],
        [pltpu.VMEM((C, PACKED), jnp.int32),
         pltpu.VMEM((C, HIDDEN), jnp.float32),
         pltpu.VMEM((C, PACKED), jnp.int32),
         pltpu.VMEM((C, PACKED), jnp.int32)],
        pltpu.VMEM((C, HIDDEN), jnp.float32),
        pltpu.VMEM((C, HIDDEN), jnp.float32),
        pltpu.SemaphoreType.DMA,
        pltpu.SemaphoreType.DMA,
        pltpu.SemaphoreType.DMA,
        pltpu.SemaphoreType.DMA,
        pltpu.SemaphoreType.DMA,
    ],
)
def _embed_sum(comb_idx, org_idx,
               spec_t, org_t, abx_t, intp_t,
               out_h, idx_v, bufs_a, bufs_b, out_a, out_b,
               gsem_a, gsem_b, isem, ssem_a, ssem_b):
    _sc_body((comb_idx, org_idx),
             (spec_t, org_t, abx_t, intp_t),
             out_h, idx_v, bufs_a, bufs_b, out_a, out_b,
             gsem_a, gsem_b, isem, ssem_a, ssem_b)


def kernel(specimen_ids, organism_ids, antibiotic_ids, interpretation_ids,
           specimen_table, organism_table, antibiotic_table, interpretation_table):
    batch, hist = specimen_ids.shape
    spec_p, abx_p, intp_p = _pack_tables(
        specimen_table, antibiotic_table, interpretation_table)
    org_p = organism_table
    shp = (NW, NCHUNK, C)
    comb = (specimen_ids | (antibiotic_ids << 10)
            | (interpretation_ids << 20)).T.reshape(shp)
    orga = (organism_ids + N_SMALL).T.reshape(shp)
    out = _embed_sum(comb, orga, spec_p, org_p, abx_p, intp_p)
    return out.reshape(hist, batch, HIDDEN).transpose(1, 0, 2)
